# baseline (speedup 1.0000x reference)
import dataclasses
import functools

import jax
import jax.numpy as jnp
from jax import lax
from jax.experimental import pallas as pl
from jax.experimental.pallas import tpu as pltpu
from jax.experimental.pallas import tpu_sc as plsc

B = 4096
F = 26
D = 32
L = 16
NC, NS = 2, 16
NW = NC * NS
ITEMS_PER_W = B // NW
CHUNK = 16
NCHUNK = ITEMS_PER_W // CHUNK
CF = CHUNK * F


def kernel(feat_index, feat_value, em1_weight, em2_weight):
    idx_flat = feat_index.reshape(-1)
    val_flat = feat_value.reshape(-1)

    mesh = plsc.VectorSubcoreMesh(core_axis_name="c", subcore_axis_name="s")

    cp = pltpu.CompilerParams()
    if "needs_layout_passes" in pltpu.CompilerParams.__dataclass_fields__:
        cp = dataclasses.replace(cp, needs_layout_passes=False)
    if "use_tc_tiling_on_sc" in pltpu.CompilerParams.__dataclass_fields__:
        cp = dataclasses.replace(cp, use_tc_tiling_on_sc=False)

    @functools.partial(
        pl.kernel,
        compiler_params=cp,
        out_type=(
            jax.ShapeDtypeStruct((B * F,), jnp.float32),
            jax.ShapeDtypeStruct((B * D,), jnp.float32),
        ),
        mesh=mesh,
        scratch_types=[
            pltpu.VMEM((CF,), jnp.int32),
            pltpu.VMEM((CF,), jnp.float32),
            pltpu.VMEM((CF, D), jnp.float32),
            pltpu.VMEM((CF,), jnp.float32),
            pltpu.VMEM((CF,), jnp.float32),
            pltpu.VMEM((CHUNK * D,), jnp.float32),
            pltpu.SemaphoreType.DMA,
            pltpu.SemaphoreType.DMA,
        ],
    )
    def fm_kernel(em1_hbm, em2_hbm, idx_hbm, val_hbm, y1_hbm, y2_hbm,
                  idx_v, val_v, rows_v, g2_v, y1_v, y2_v, sem1, sem2):
        wid = lax.axis_index("s") * NC + lax.axis_index("c")

        @pl.loop(0, NCHUNK)
        def _chunk(c):
            base = wid * (ITEMS_PER_W * F) + c * CF
            pltpu.sync_copy(idx_hbm.at[pl.ds(base, CF)], idx_v)
            pltpu.sync_copy(val_hbm.at[pl.ds(base, CF)], val_v)
            cp1 = pltpu.async_copy(em1_hbm.at[idx_v], rows_v, sem1)
            cp2 = pltpu.async_copy(em2_hbm.at[idx_v], g2_v, sem2)
            cp1.wait()
            cp2.wait()

            @pl.loop(0, CHUNK)
            def _item(i):
                r0 = i * F
                s = [jnp.zeros((L,), jnp.float32) for _ in range(D // L)]
                q = [jnp.zeros((L,), jnp.float32) for _ in range(D // L)]
                for f in range(F):
                    r = r0 + f
                    vb = plsc.load_gather(
                        val_v, [jnp.full((L,), r, dtype=jnp.int32)])
                    for h in range(D // L):
                        e = rows_v[r, pl.ds(h * L, L)]
                        t = e * vb
                        s[h] = s[h] + t
                        q[h] = q[h] + t * t
                for h in range(D // L):
                    y2_v[pl.ds(i * D + h * L, L)] = 0.5 * (s[h] * s[h] - q[h])

            @pl.loop(0, CF, step=L)
            def _fo(j):
                y1_v[pl.ds(j, L)] = g2_v[pl.ds(j, L)] * val_v[pl.ds(j, L)]

            pltpu.sync_copy(y1_v, y1_hbm.at[pl.ds(base, CF)])
            ob = wid * (ITEMS_PER_W * D) + c * CHUNK * D
            pltpu.sync_copy(y2_v, y2_hbm.at[pl.ds(ob, CHUNK * D)])

    y1f, y2f = fm_kernel(em1_weight, em2_weight.reshape(-1), idx_flat, val_flat)
    return y1f.reshape(B, F), y2f.reshape(B, D)
